# baseline (speedup 1.0000x reference)
import functools

import jax
import jax.numpy as jnp
from jax.experimental import pallas as pl
from jax.experimental.pallas import tpu as pltpu


def _mm_body(x_ref, we_ref, wo_ref, o_ref, x16_ref, *, n_blk):
    @pl.when(pl.program_id(0) == 0)
    def _():
        x16_ref[...] = x_ref[...].astype(jnp.bfloat16)

    x16 = x16_ref[...]
    dn = (((1,), (1,)), ((), ()))
    o_ref[:, :n_blk] = jax.lax.dot_general(
        x16, we_ref[...].astype(jnp.bfloat16), dimension_numbers=dn,
        preferred_element_type=jnp.float32)
    o_ref[:, n_blk:] = jax.lax.dot_general(
        x16, wo_ref[...].astype(jnp.bfloat16), dimension_numbers=dn,
        preferred_element_type=jnp.float32)


@functools.partial(jax.jit, static_argnames=("n_blk",))
def _spmm(x, W, n_blk=128):
    m, kdim = x.shape
    ndim = W.shape[0]
    return pl.pallas_call(
        functools.partial(_mm_body, n_blk=n_blk),
        grid=(ndim // (2 * n_blk),),
        in_specs=[
            pl.BlockSpec((m, kdim), lambda n: (0, 0)),
            pl.BlockSpec((n_blk, kdim), lambda n: (2 * n, 0)),
            pl.BlockSpec((n_blk, kdim), lambda n: (2 * n + 1, 0)),
        ],
        out_specs=pl.BlockSpec((m, 2 * n_blk), lambda n: (0, n)),
        out_shape=jax.ShapeDtypeStruct((m, ndim), jnp.float32),
        scratch_shapes=[pltpu.VMEM((m, kdim), jnp.bfloat16)],
        compiler_params=pltpu.CompilerParams(
            dimension_semantics=("arbitrary",)),
    )(x, W, W)


def kernel(x, W, bias):
    del bias
    return _spmm(x, W)


# speedup vs baseline: 1.0525x; 1.0525x over previous
import functools

import jax
import jax.numpy as jnp
from jax.experimental import pallas as pl
from jax.experimental.pallas import tpu as pltpu


def _mm_body(x_ref, w_ref, o_ref, x16_ref):
    @pl.when(pl.program_id(0) == 0)
    def _():
        x16_ref[...] = x_ref[...].astype(jnp.bfloat16)

    w_blk = w_ref[...].astype(jnp.bfloat16)
    o_ref[...] = jax.lax.dot_general(
        x16_ref[...], w_blk,
        dimension_numbers=(((1,), (1,)), ((), ())),
        preferred_element_type=jnp.float32)


@functools.partial(jax.jit, static_argnames=("n_blk",))
def _spmm(x, W, n_blk=256):
    m, kdim = x.shape
    ndim = W.shape[0]
    return pl.pallas_call(
        _mm_body,
        grid=(ndim // n_blk,),
        in_specs=[
            pl.BlockSpec((m, kdim), lambda n: (0, 0)),
            pl.BlockSpec((n_blk, kdim), lambda n: (n, 0)),
        ],
        out_specs=pl.BlockSpec((m, n_blk), lambda n: (0, n)),
        out_shape=jax.ShapeDtypeStruct((m, ndim), jnp.float32),
        scratch_shapes=[pltpu.VMEM((m, kdim), jnp.bfloat16)],
        compiler_params=pltpu.CompilerParams(
            dimension_semantics=("arbitrary",)),
    )(x, W)


def kernel(x, W, bias):
    del bias
    return _spmm(x, W)
